# Initial kernel scaffold; baseline (speedup 1.0000x reference)
import jax
import jax.numpy as jnp
from jax.experimental import pallas as pl


def kernel(features, edge_weight, eps, Ws, bs, edge_index):
    raise NotImplementedError("write your pallas kernel here")


# speedup vs baseline: 6.4532x; 6.4532x over previous
import functools

import jax
import jax.numpy as jnp
from jax import lax
from jax.experimental import pallas as pl
from jax.experimental.pallas import tpu as pltpu
from jax.experimental.pallas import tpu_sc as plsc

B, N, D, H = 16, 640, 128, 128
BN = B * N
E = 327680
NC, NS, LANES = 2, 16, 16
NW = NC * NS
EPW = E // NW
C = 128
NCHUNK = EPW // C
ROWS_PER_TILE = BN // NS


def _agg_kernel(h_hbm, src_hbm, dst_hbm, w_hbm, out_hbm,
                src_v, dst_v, w_v, rows_v, acc_sh, sem):
  cid = lax.axis_index("c")
  sid = lax.axis_index("s")
  wid = sid * NC + cid

  zeros16 = jnp.zeros((LANES,), jnp.float32)

  def zrow(r, carry):
    for j in range(H // LANES):
      rows_v[r, pl.ds(j * LANES, LANES)] = zeros16
    return carry

  lax.fori_loop(0, C, zrow, 0)
  for t in range(ROWS_PER_TILE // C):
    pltpu.sync_copy(rows_v, acc_sh.at[pl.ds(sid * ROWS_PER_TILE + t * C, C)])
  plsc.subcore_barrier()

  base = wid * EPW

  def chunk(i, carry):
    off = base + i * C
    pltpu.sync_copy(src_hbm.at[pl.ds(off, C)], src_v)
    pltpu.sync_copy(dst_hbm.at[pl.ds(off, C)], dst_v)
    pltpu.sync_copy(w_hbm.at[pl.ds(off, C)], w_v)
    pltpu.async_copy(h_hbm.at[src_v], rows_v, sem).wait()

    def sgroup(g, carry2):
      wg = w_v[pl.ds(g * LANES, LANES)]
      for e in range(LANES):
        wbr = jnp.full((LANES,), wg[e])
        r = g * LANES + e
        for j in range(H // LANES):
          sl = pl.ds(j * LANES, LANES)
          rows_v[r, sl] = rows_v[r, sl] * wbr
      return carry2

    lax.fori_loop(0, C // LANES, sgroup, 0)
    pltpu.sync_copy(rows_v, acc_sh.at[dst_v], add=True)
    return carry

  lax.fori_loop(0, NCHUNK, chunk, 0)
  plsc.subcore_barrier()
  slab = pl.ds(sid * ROWS_PER_TILE, ROWS_PER_TILE)
  pltpu.sync_copy(acc_sh.at[slab], out_hbm.at[cid, slab])


def _aggregate(h, src, dst, w):
  mesh = plsc.VectorSubcoreMesh(core_axis_name="c", subcore_axis_name="s",
                                num_cores=NC, num_subcores=NS)
  return pl.kernel(
      _agg_kernel,
      out_type=jax.ShapeDtypeStruct((NC, BN, H), jnp.float32),
      mesh=mesh,
      scratch_types=[
          pltpu.VMEM((C,), jnp.int32),
          pltpu.VMEM((C,), jnp.int32),
          pltpu.VMEM((C,), jnp.float32),
          pltpu.VMEM((C, H), jnp.float32),
          pltpu.VMEM_SHARED((BN, H), jnp.float32),
          pltpu.SemaphoreType.DMA,
      ],
  )(h, src, dst, w)


def _mlp0_body(x_ref, w1_ref, b1_ref, w2_ref, b2_ref, h_ref, pool_ref):
  t = jnp.dot(x_ref[...], w1_ref[...], preferred_element_type=jnp.float32)
  t = jnp.maximum(t + b1_ref[...], 0.0)
  t = jnp.dot(t, w2_ref[...], preferred_element_type=jnp.float32) + b2_ref[...]
  h_ref[...] = t
  pool_ref[0] = jnp.max(t, axis=0, keepdims=True)


def _mlp0(x, W1, b1, W2, b2):
  return pl.pallas_call(
      _mlp0_body,
      grid=(B,),
      in_specs=[
          pl.BlockSpec((N, D), lambda i: (i, 0)),
          pl.BlockSpec((D, H), lambda i: (0, 0)),
          pl.BlockSpec((1, H), lambda i: (0, 0)),
          pl.BlockSpec((H, H), lambda i: (0, 0)),
          pl.BlockSpec((1, H), lambda i: (0, 0)),
      ],
      out_specs=[
          pl.BlockSpec((N, H), lambda i: (i, 0)),
          pl.BlockSpec((1, 1, H), lambda i: (i, 0, 0)),
      ],
      out_shape=[
          jax.ShapeDtypeStruct((BN, H), jnp.float32),
          jax.ShapeDtypeStruct((B, 1, H), jnp.float32),
      ],
  )(x, W1, b1, W2, b2)


def _mlp_body(p_ref, h_ref, s_ref, w1_ref, b1_ref, w2_ref, b2_ref,
              hout_ref, pool_ref):
  x = p_ref[0] + p_ref[1] + s_ref[0, 0] * h_ref[...]
  t = jnp.dot(x, w1_ref[...], preferred_element_type=jnp.float32)
  t = jnp.maximum(t + b1_ref[...], 0.0)
  t = jnp.dot(t, w2_ref[...], preferred_element_type=jnp.float32) + b2_ref[...]
  hout_ref[...] = t
  pool_ref[0] = jnp.max(t, axis=0, keepdims=True)


def _mlp_layer(partials, h, scale, W1, b1, W2, b2):
  return pl.pallas_call(
      _mlp_body,
      grid=(B,),
      in_specs=[
          pl.BlockSpec((NC, N, H), lambda i: (0, i, 0)),
          pl.BlockSpec((N, H), lambda i: (i, 0)),
          pl.BlockSpec(memory_space=pltpu.SMEM),
          pl.BlockSpec((H, H), lambda i: (0, 0)),
          pl.BlockSpec((1, H), lambda i: (0, 0)),
          pl.BlockSpec((H, H), lambda i: (0, 0)),
          pl.BlockSpec((1, H), lambda i: (0, 0)),
      ],
      out_specs=[
          pl.BlockSpec((N, H), lambda i: (i, 0)),
          pl.BlockSpec((1, 1, H), lambda i: (i, 0, 0)),
      ],
      out_shape=[
          jax.ShapeDtypeStruct((BN, H), jnp.float32),
          jax.ShapeDtypeStruct((B, 1, H), jnp.float32),
      ],
  )(partials, h, scale, W1, b1, W2, b2)


def kernel(features, edge_weight, eps, Ws, bs, edge_index):
  src = edge_index[0]
  dst = edge_index[1]
  x = features.reshape(BN, D)
  pools = []
  h, pool = _mlp0(x, Ws[0][0], bs[0][0].reshape(1, H),
                  Ws[0][1], bs[0][1].reshape(1, H))
  pools.append(pool)
  for i in range(len(Ws) - 1):
    partials = _aggregate(h, src, dst, edge_weight)
    scale = (1.0 + eps[i]).reshape(1, 1)
    h, pool = _mlp_layer(partials, h, scale,
                         Ws[i + 1][0], bs[i + 1][0].reshape(1, H),
                         Ws[i + 1][1], bs[i + 1][1].reshape(1, H))
    pools.append(pool)
  out = jnp.concatenate(pools, axis=1)
  return out.reshape(B, len(Ws) * H)


# speedup vs baseline: 13.9045x; 2.1547x over previous
import functools

import jax
import jax.numpy as jnp
from jax import lax
from jax.experimental import pallas as pl
from jax.experimental.pallas import tpu as pltpu
from jax.experimental.pallas import tpu_sc as plsc

B, N, D, H = 16, 640, 128, 128
BN = B * N
E = 327680
NC, NS, LANES = 2, 16, 16
NW = NC * NS
EPW = E // NW
C = 128
NCHUNK = EPW // C
ROWS_PER_TILE = BN // NS


G = 8
NBLK = NCHUNK // G


def _agg_kernel(h_hbm, src_hbm, dst_hbm, w_hbm, out_hbm,
                sidx, didx, wblk, rows2, acc_sh, gsem, isem):
  cid = lax.axis_index("c")
  sid = lax.axis_index("s")
  wid = sid * NC + cid

  def idx_start(blk, s):
    pltpu.async_copy(src_hbm.at[wid, blk], sidx.at[s], isem)
    pltpu.async_copy(dst_hbm.at[wid, blk], didx.at[s], isem)
    pltpu.async_copy(w_hbm.at[wid, blk], wblk.at[s], isem)

  def idx_wait(blk, s):
    pltpu.make_async_copy(src_hbm.at[wid, blk], sidx.at[s], isem).wait()
    pltpu.make_async_copy(dst_hbm.at[wid, blk], didx.at[s], isem).wait()
    pltpu.make_async_copy(w_hbm.at[wid, blk], wblk.at[s], isem).wait()

  idx_start(0, 0)

  zeros16 = jnp.zeros((LANES,), jnp.float32)

  def zrow(r, carry):
    for j in range(H // LANES):
      rows2[0, r, pl.ds(j * LANES, LANES)] = zeros16
    return carry

  lax.fori_loop(0, C, zrow, 0)
  for t in range(ROWS_PER_TILE // C):
    pltpu.sync_copy(rows2.at[0],
                    acc_sh.at[pl.ds(sid * ROWS_PER_TILE + t * C, C)])
  plsc.subcore_barrier()

  def gather_start(s, c, b):
    pltpu.async_copy(h_hbm.at[sidx.at[s, c]], rows2.at[b], gsem)

  def gather_wait(s, c, b):
    pltpu.make_async_copy(h_hbm.at[sidx.at[s, c]], rows2.at[b], gsem).wait()

  def scale_scatter(s, c, b):
    def sgroup(g, carry2):
      wg = wblk[s, c, pl.ds(g * LANES, LANES)]
      for e in range(LANES):
        wbr = jnp.full((LANES,), wg[e])
        r = g * LANES + e
        for j in range(H // LANES):
          sl = pl.ds(j * LANES, LANES)
          rows2[b, r, sl] = rows2[b, r, sl] * wbr
      return carry2

    lax.fori_loop(0, C // LANES, sgroup, 0)
    pltpu.sync_copy(rows2.at[b], acc_sh.at[didx.at[s, c]], add=True)

  def block(blk, carry):
    s = lax.rem(blk, 2)
    idx_wait(blk, s)

    @pl.when(blk + 1 < NBLK)
    def _():
      idx_start(blk + 1, 1 - s)

    gather_start(s, 0, 0)

    def pair(j, carry2):
      c0 = 2 * j
      c1 = c0 + 1
      gather_start(s, c1, 1)
      gather_wait(s, c0, 0)
      scale_scatter(s, c0, 0)

      @pl.when(c1 + 1 < G)
      def _():
        gather_start(s, c1 + 1, 0)

      gather_wait(s, c1, 1)
      scale_scatter(s, c1, 1)
      return carry2

    lax.fori_loop(0, G // 2, pair, 0)
    return carry

  lax.fori_loop(0, NBLK, block, 0)
  plsc.subcore_barrier()
  slab = pl.ds(sid * ROWS_PER_TILE, ROWS_PER_TILE)
  pltpu.sync_copy(acc_sh.at[slab], out_hbm.at[cid, slab])


def _aggregate(h, src, dst, w):
  mesh = plsc.VectorSubcoreMesh(core_axis_name="c", subcore_axis_name="s",
                                num_cores=NC, num_subcores=NS)
  return pl.kernel(
      _agg_kernel,
      out_type=jax.ShapeDtypeStruct((NC, BN, H), jnp.float32),
      mesh=mesh,
      scratch_types=[
          pltpu.VMEM((2, G, C), jnp.int32),
          pltpu.VMEM((2, G, C), jnp.int32),
          pltpu.VMEM((2, G, C), jnp.float32),
          pltpu.VMEM((2, C, H), jnp.float32),
          pltpu.VMEM_SHARED((BN, H), jnp.float32),
          pltpu.SemaphoreType.DMA,
          pltpu.SemaphoreType.DMA,
      ],
  )(h, src.reshape(NW, NBLK, G, C),
    dst.reshape(NW, NBLK, G, C), w.reshape(NW, NBLK, G, C))


def _mlp0_body(x_ref, w1_ref, b1_ref, w2_ref, b2_ref, h_ref, pool_ref):
  t = jnp.dot(x_ref[...], w1_ref[...], preferred_element_type=jnp.float32)
  t = jnp.maximum(t + b1_ref[...], 0.0)
  t = jnp.dot(t, w2_ref[...], preferred_element_type=jnp.float32) + b2_ref[...]
  h_ref[...] = t
  pool_ref[0] = jnp.max(t, axis=0, keepdims=True)


def _mlp0(x, W1, b1, W2, b2):
  return pl.pallas_call(
      _mlp0_body,
      grid=(B,),
      in_specs=[
          pl.BlockSpec((N, D), lambda i: (i, 0)),
          pl.BlockSpec((D, H), lambda i: (0, 0)),
          pl.BlockSpec((1, H), lambda i: (0, 0)),
          pl.BlockSpec((H, H), lambda i: (0, 0)),
          pl.BlockSpec((1, H), lambda i: (0, 0)),
      ],
      out_specs=[
          pl.BlockSpec((N, H), lambda i: (i, 0)),
          pl.BlockSpec((1, 1, H), lambda i: (i, 0, 0)),
      ],
      out_shape=[
          jax.ShapeDtypeStruct((BN, H), jnp.float32),
          jax.ShapeDtypeStruct((B, 1, H), jnp.float32),
      ],
  )(x, W1, b1, W2, b2)


def _mlp_body(p_ref, h_ref, s_ref, w1_ref, b1_ref, w2_ref, b2_ref,
              hout_ref, pool_ref):
  x = p_ref[0] + p_ref[1] + s_ref[0, 0] * h_ref[...]
  t = jnp.dot(x, w1_ref[...], preferred_element_type=jnp.float32)
  t = jnp.maximum(t + b1_ref[...], 0.0)
  t = jnp.dot(t, w2_ref[...], preferred_element_type=jnp.float32) + b2_ref[...]
  hout_ref[...] = t
  pool_ref[0] = jnp.max(t, axis=0, keepdims=True)


def _mlp_layer(partials, h, scale, W1, b1, W2, b2):
  return pl.pallas_call(
      _mlp_body,
      grid=(B,),
      in_specs=[
          pl.BlockSpec((NC, N, H), lambda i: (0, i, 0)),
          pl.BlockSpec((N, H), lambda i: (i, 0)),
          pl.BlockSpec(memory_space=pltpu.SMEM),
          pl.BlockSpec((H, H), lambda i: (0, 0)),
          pl.BlockSpec((1, H), lambda i: (0, 0)),
          pl.BlockSpec((H, H), lambda i: (0, 0)),
          pl.BlockSpec((1, H), lambda i: (0, 0)),
      ],
      out_specs=[
          pl.BlockSpec((N, H), lambda i: (i, 0)),
          pl.BlockSpec((1, 1, H), lambda i: (i, 0, 0)),
      ],
      out_shape=[
          jax.ShapeDtypeStruct((BN, H), jnp.float32),
          jax.ShapeDtypeStruct((B, 1, H), jnp.float32),
      ],
  )(partials, h, scale, W1, b1, W2, b2)


def kernel(features, edge_weight, eps, Ws, bs, edge_index):
  src = edge_index[0]
  dst = edge_index[1]
  x = features.reshape(BN, D)
  pools = []
  h, pool = _mlp0(x, Ws[0][0], bs[0][0].reshape(1, H),
                  Ws[0][1], bs[0][1].reshape(1, H))
  pools.append(pool)
  for i in range(len(Ws) - 1):
    partials = _aggregate(h, src, dst, edge_weight)
    scale = (1.0 + eps[i]).reshape(1, 1)
    h, pool = _mlp_layer(partials, h, scale,
                         Ws[i + 1][0], bs[i + 1][0].reshape(1, H),
                         Ws[i + 1][1], bs[i + 1][1].reshape(1, H))
    pools.append(pool)
  out = jnp.concatenate(pools, axis=1)
  return out.reshape(B, len(Ws) * H)
